# baseline (speedup 1.0000x reference)
import functools

import jax
import jax.numpy as jnp
from jax import lax
from jax.experimental import pallas as pl
from jax.experimental.pallas import tpu as pltpu
from jax.experimental.pallas import tpu_sc as plsc

B, L, T = 1024, 200, 201
DOM_D, FEAT_D, D = 96, 32, 128
TABLE_ROWS = 5162

NC, NS, LANES = 2, 16, 16
NW = NC * NS
RPW = B // NW
IDX_CHUNK = 112
NPAD = 2 * IDX_CHUNK


def _sc_body(x_hbm, tab_hbm, w_hbm, bias_hbm, cls_hbm, out_hbm,
             xv, idxv, rowsv, featv, wv, biasv, clsv, sem):
    wid = lax.axis_index("s") * NC + lax.axis_index("c")

    pltpu.sync_copy(w_hbm, wv)
    pltpu.sync_copy(bias_hbm, biasv)
    pltpu.sync_copy(cls_hbm, clsv)

    w00 = wv[0, pl.ds(0, 16)]
    w01 = wv[0, pl.ds(16, 16)]
    w10 = wv[1, pl.ds(0, 16)]
    w11 = wv[1, pl.ds(16, 16)]
    w20 = wv[2, pl.ds(0, 16)]
    w21 = wv[2, pl.ds(16, 16)]
    b0 = biasv[pl.ds(0, 16)]
    b1 = biasv[pl.ds(16, 16)]

    lane4 = lax.iota(jnp.int32, 16) * 4 + 3

    def row_body(i, carry):
        bi = wid * RPW + i
        pltpu.sync_copy(x_hbm.at[bi], xv.at[pl.ds(0, L * 4)])

        for c in range(NPAD // 16):
            fidx = plsc.load_gather(xv, [lane4 + c * 64])
            ivec = jnp.clip(fidx.astype(jnp.int32), 0, TABLE_ROWS - 1)
            j, col = (16 * c) // IDX_CHUNK, (16 * c) % IDX_CHUNK
            idxv[j, pl.ds(col, 16)] = ivec

        cp0 = pltpu.async_copy(
            tab_hbm.at[idxv.at[0]], rowsv.at[pl.ds(0, IDX_CHUNK)], sem)
        cp1 = pltpu.async_copy(
            tab_hbm.at[idxv.at[1]], rowsv.at[pl.ds(IDX_CHUNK, IDX_CHUNK)], sem)

        def tok_body(t, carry2):
            v = xv[pl.ds(4 * t, 16)]
            s0, s1, s2 = v[0], v[1], v[2]
            featv[t, pl.ds(0, 16)] = s0 * w00 + s1 * w10 + s2 * w20 + b0
            featv[t, pl.ds(16, 16)] = s0 * w01 + s1 * w11 + s2 * w21 + b1
            return carry2

        lax.fori_loop(0, L, tok_body, 0)
        cp0.wait()
        cp1.wait()

        pltpu.sync_copy(clsv, out_hbm.at[bi, 0])
        pltpu.sync_copy(rowsv.at[pl.ds(0, L)],
                        out_hbm.at[bi, pl.ds(1, L), pl.ds(0, DOM_D)])
        pltpu.sync_copy(featv,
                        out_hbm.at[bi, pl.ds(1, L), pl.ds(DOM_D, FEAT_D)])
        return carry

    lax.fori_loop(0, RPW, row_body, 0)


def _mask_body(l_ref, o_ref):
    t = lax.broadcasted_iota(jnp.int32, (B, T), 1)
    lb = l_ref[:]
    o_ref[:] = (t > lb) & (t != 0)


@jax.jit
def kernel(x, l, dom_table, W, b, cls):
    xf = x.reshape(B, L * 4)
    cls_flat = cls.reshape(D)
    l2 = l.astype(jnp.int32).reshape(B, 1)

    mesh = plsc.VectorSubcoreMesh(
        core_axis_name="c", subcore_axis_name="s",
        num_cores=NC, num_subcores=NS)

    sc_fn = pl.kernel(
        _sc_body,
        out_type=jax.ShapeDtypeStruct((B, T, D), jnp.float32),
        mesh=mesh,
        scratch_types=[
            pltpu.VMEM((NPAD * 4,), jnp.float32),
            pltpu.VMEM((2, IDX_CHUNK), jnp.int32),
            pltpu.VMEM((NPAD, DOM_D), jnp.float32),
            pltpu.VMEM((L, FEAT_D), jnp.float32),
            pltpu.VMEM((3, FEAT_D), jnp.float32),
            pltpu.VMEM((FEAT_D,), jnp.float32),
            pltpu.VMEM((D,), jnp.float32),
            pltpu.SemaphoreType.DMA,
        ],
        compiler_params=pltpu.CompilerParams(
            use_tc_tiling_on_sc=False, needs_layout_passes=False),
    )
    full_embedding = sc_fn(xf, dom_table, W, b, cls_flat)

    padding_mask = pl.pallas_call(
        _mask_body,
        out_shape=jax.ShapeDtypeStruct((B, T), jnp.bool_),
    )(l2)

    return (full_embedding, padding_mask)
